# baseline (device time: 12543 ns/iter reference)
import jax
import jax.numpy as jnp
from jax import lax
from jax.experimental import pallas as pl
from jax.experimental.pallas import tpu as pltpu

NZ = 4
CAP = 160


def kernel(x, dest):
    m, n = x.shape
    dest2 = dest.reshape(1, m)

    def body(x_ref, d_ref, out_ref, dg, sbuf, rbuf,
             dsend, drecv, bsend, brecv):
        my_x = lax.axis_index("x")
        my_y = lax.axis_index("y")
        my_z = lax.axis_index("z")

        barrier = pltpu.get_barrier_semaphore()
        for d in range(1, NZ):
            pz = lax.rem(my_z + d, NZ)
            pl.semaphore_signal(
                barrier, inc=1,
                device_id=(my_x, my_y, pz),
                device_id_type=pl.DeviceIdType.MESH,
            )
        pl.semaphore_wait(barrier, NZ - 1)

        sends = []

        dg[my_z] = d_ref[...]
        for d in range(1, NZ):
            pz = lax.rem(my_z + d, NZ)
            rd = pltpu.make_async_remote_copy(
                src_ref=dg.at[my_z], dst_ref=dg.at[my_z],
                send_sem=dsend.at[d - 1], recv_sem=drecv.at[my_z],
                device_id=(my_x, my_y, pz),
                device_id_type=pl.DeviceIdType.MESH,
            )
            rd.start()
            sends.append(rd)

        xb = x_ref[...].astype(jnp.bfloat16)
        jvals = lax.broadcasted_iota(jnp.int32, (NZ, m), 0)
        maskl = (d_ref[...] == jvals).astype(jnp.int32)

        a = lax.broadcasted_iota(jnp.int32, (m, m), 0)
        b = lax.broadcasted_iota(jnp.int32, (m, m), 1)
        tri = (a <= b).astype(jnp.bfloat16)
        csl = lax.dot_general(
            maskl.astype(jnp.bfloat16), tri, (((1,), (0,)), ((), ())),
            preferred_element_type=jnp.float32,
        )

        riota = lax.broadcasted_iota(jnp.int32, (CAP, m), 0)

        csl_i = csl.astype(jnp.int32)
        zio = lax.broadcasted_iota(jnp.int32, (NZ, m), 0)
        for d in range(NZ - 1, 0, -1):
            pz = lax.rem(my_z + d, NZ)
            hot = zio == pz
            cslr = jnp.sum(
                jnp.where(hot, csl_i, 0), axis=0, keepdims=True
            )
            maskr = jnp.sum(
                jnp.where(hot, maskl, 0), axis=0, keepdims=True
            )
            sel = (cslr - 1 == riota) & (maskr > 0)
            sj = jnp.where(sel, 1.0, 0.0).astype(jnp.bfloat16)
            sbuf[pz] = lax.dot_general(
                sj, xb, (((1,), (0,)), ((), ())),
                preferred_element_type=jnp.float32,
            ).astype(jnp.bfloat16)
            rb = pltpu.make_async_remote_copy(
                src_ref=sbuf.at[pz], dst_ref=rbuf.at[my_z],
                send_sem=bsend.at[d - 1], recv_sem=brecv.at[my_z],
                device_id=(my_x, my_y, pz),
                device_id_type=pl.DeviceIdType.MESH,
            )
            rb.start()
            sends.append(rb)

        hot = zio == my_z
        cslr = jnp.sum(jnp.where(hot, csl_i, 0), axis=0, keepdims=True)
        maskr = jnp.sum(jnp.where(hot, maskl, 0), axis=0, keepdims=True)
        sel = (cslr - 1 == riota) & (maskr > 0)
        sj = jnp.where(sel, 1.0, 0.0).astype(jnp.bfloat16)
        rbuf[my_z] = lax.dot_general(
            sj, xb, (((1,), (0,)), ((), ())),
            preferred_element_type=jnp.float32,
        ).astype(jnp.bfloat16)

        for d in range(1, NZ):
            sz = lax.rem(my_z - d + NZ, NZ)
            wd = pltpu.make_async_remote_copy(
                src_ref=dg.at[sz], dst_ref=dg.at[sz],
                send_sem=dsend.at[d - 1], recv_sem=drecv.at[sz],
                device_id=(my_x, my_y, sz),
                device_id_type=pl.DeviceIdType.MESH,
            )
            wd.wait_recv()

        dall = dg[:, 0, :]
        mfa = (dall == my_z).astype(jnp.float32)
        cnt = jnp.sum(mfa, axis=1).astype(jnp.int32)

        kio = lax.broadcasted_iota(jnp.int32, (m, CAP), 0)
        rio = lax.broadcasted_iota(jnp.int32, (m, CAP), 1)
        pis = []
        base = jnp.int32(0)
        for i in range(NZ):
            pis.append(jnp.where(
                (kio == base + rio) & (rio < cnt[i]), 1.0, 0.0
            ).astype(jnp.bfloat16))
            base = base + cnt[i]

        for d in range(1, NZ):
            sz = lax.rem(my_z - d + NZ, NZ)
            wb = pltpu.make_async_remote_copy(
                src_ref=rbuf.at[sz], dst_ref=rbuf.at[sz],
                send_sem=bsend.at[d - 1], recv_sem=brecv.at[sz],
                device_id=(my_x, my_y, sz),
                device_id_type=pl.DeviceIdType.MESH,
            )
            wb.wait_recv()

        acc = jnp.zeros((m, n), jnp.float32)
        for i in range(NZ):
            acc = acc + lax.dot_general(
                pis[i], rbuf[i], (((1,), (0,)), ((), ())),
                preferred_element_type=jnp.float32,
            )
        out_ref[...] = acc

        for s in sends:
            s.wait_send()

    return pl.pallas_call(
        body,
        out_shape=jax.ShapeDtypeStruct((m, n), jnp.float32),
        in_specs=[
            pl.BlockSpec(memory_space=pltpu.VMEM),
            pl.BlockSpec(memory_space=pltpu.VMEM),
        ],
        out_specs=pl.BlockSpec(memory_space=pltpu.VMEM),
        scratch_shapes=[
            pltpu.VMEM((NZ, 1, m), jnp.int32),
            pltpu.VMEM((NZ, CAP, n), jnp.bfloat16),
            pltpu.VMEM((NZ, CAP, n), jnp.bfloat16),
            pltpu.SemaphoreType.DMA((NZ - 1,)),
            pltpu.SemaphoreType.DMA((NZ,)),
            pltpu.SemaphoreType.DMA((NZ - 1,)),
            pltpu.SemaphoreType.DMA((NZ,)),
        ],
        compiler_params=pltpu.CompilerParams(collective_id=0),
    )(x, dest2)


# device time: 9132 ns/iter; 1.3735x vs baseline; 1.3735x over previous
import os

import jax
import jax.numpy as jnp
from jax import lax
from jax.experimental import pallas as pl
from jax.experimental.pallas import tpu as pltpu

COMM = os.environ.get("A2AV_NO_COMM") != "1"
NZ = 4
CAP = 160


def kernel(x, dest):
    m, n = x.shape
    dest2 = dest.reshape(1, m)

    def body(x_ref, d_ref, out_ref, xv, dg, sbuf, rbuf,
             iosem, dsend, drecv, bsend, brecv):
        my_x = lax.axis_index("x")
        my_y = lax.axis_index("y")
        my_z = lax.axis_index("z")

        cp_x = pltpu.make_async_copy(x_ref, xv, iosem.at[0])
        cp_d = pltpu.make_async_copy(d_ref, dg.at[my_z], iosem.at[1])
        cp_x.start()
        cp_d.start()

        if COMM:
            barrier = pltpu.get_barrier_semaphore()
            for d in range(1, NZ):
                pz = lax.rem(my_z + d, NZ)
                pl.semaphore_signal(
                    barrier, inc=1,
                    device_id=(my_x, my_y, pz),
                    device_id_type=pl.DeviceIdType.MESH,
                )

        low = my_z <= 1
        peers = [
            jnp.where(low, 3, 0),
            jnp.where(low, 2, 1),
            jnp.where(low, 1 - my_z, 5 - my_z),
        ]

        cp_d.wait()
        cp_x.wait()

        xb = xv[...].astype(jnp.bfloat16)
        jvals = lax.broadcasted_iota(jnp.int32, (NZ, m), 0)
        maskl = dg[my_z] == jvals

        a = lax.broadcasted_iota(jnp.int32, (m, m), 0)
        b = lax.broadcasted_iota(jnp.int32, (m, m), 1)
        tri = (a <= b).astype(jnp.bfloat16)
        csl = lax.dot_general(
            maskl.astype(jnp.bfloat16), tri, (((1,), (0,)), ((), ())),
            preferred_element_type=jnp.float32,
        ).astype(jnp.int32)
        mval = jnp.where(maskl, csl, 0)

        zio = lax.broadcasted_iota(jnp.int32, (NZ, m), 0)
        riota = lax.broadcasted_iota(jnp.int32, (CAP, m), 0)

        def build_block(pz):
            mvalr = jnp.sum(
                jnp.where(zio == pz, mval, 0), axis=0, keepdims=True
            )
            sj = (mvalr - 1 == riota).astype(jnp.bfloat16)
            return lax.dot_general(
                sj, xb, (((1,), (0,)), ((), ())),
                preferred_element_type=jnp.float32,
            ).astype(jnp.bfloat16)

        blk0 = build_block(peers[0])

        if COMM:
            pl.semaphore_wait(barrier, NZ - 1)

        sends = []
        if COMM:
            for k, pz in enumerate(peers):
                rd = pltpu.make_async_remote_copy(
                    src_ref=dg.at[my_z], dst_ref=dg.at[my_z],
                    send_sem=dsend.at[k], recv_sem=drecv.at[my_z],
                    device_id=(my_x, my_y, pz),
                    device_id_type=pl.DeviceIdType.MESH,
                )
                rd.start()
                sends.append(rd)

        for k, pz in enumerate(peers):
            sbuf[pz] = blk0 if k == 0 else build_block(pz)
            if COMM:
                rb = pltpu.make_async_remote_copy(
                    src_ref=sbuf.at[pz], dst_ref=rbuf.at[my_z],
                    send_sem=bsend.at[k], recv_sem=brecv.at[my_z],
                    device_id=(my_x, my_y, pz),
                    device_id_type=pl.DeviceIdType.MESH,
                )
                rb.start()
                sends.append(rb)

        rbuf[my_z] = build_block(my_z)

        if COMM:
            for k, pz in enumerate(peers):
                wd = pltpu.make_async_remote_copy(
                    src_ref=dg.at[pz], dst_ref=dg.at[pz],
                    send_sem=dsend.at[k], recv_sem=drecv.at[pz],
                    device_id=(my_x, my_y, pz),
                    device_id_type=pl.DeviceIdType.MESH,
                )
                wd.wait_recv()

        dall = dg[:, 0, :]
        mfa = (dall == my_z).astype(jnp.float32)
        cnt = jnp.sum(mfa, axis=1).astype(jnp.int32)

        kio = lax.broadcasted_iota(jnp.int32, (m, NZ * CAP), 0)
        rio = lax.broadcasted_iota(jnp.int32, (m, NZ * CAP), 1)
        base = jnp.int32(0)
        pos = jnp.full((1, NZ * CAP), -1, jnp.int32)
        for i in range(NZ):
            seg = (rio[0:1] >= i * CAP) & (rio[0:1] < i * CAP + cnt[i])
            pos = jnp.where(seg, rio[0:1] - i * CAP + base, pos)
            base = base + cnt[i]
        pmat = (kio == pos).astype(jnp.bfloat16)

        if COMM:
            for k, pz in enumerate(peers):
                wb = pltpu.make_async_remote_copy(
                    src_ref=rbuf.at[pz], dst_ref=rbuf.at[pz],
                    send_sem=bsend.at[k], recv_sem=brecv.at[pz],
                    device_id=(my_x, my_y, pz),
                    device_id_type=pl.DeviceIdType.MESH,
                )
                wb.wait_recv()

        blocks = rbuf[...].reshape(NZ * CAP, n)
        out_ref[...] = lax.dot_general(
            pmat, blocks, (((1,), (0,)), ((), ())),
            preferred_element_type=jnp.float32,
        )

        if COMM:
            for s in sends:
                s.wait_send()

    x = pltpu.with_memory_space_constraint(x, pltpu.MemorySpace.HBM)
    dest2 = pltpu.with_memory_space_constraint(dest2, pltpu.MemorySpace.HBM)
    return pl.pallas_call(
        body,
        out_shape=jax.ShapeDtypeStruct((m, n), jnp.float32),
        in_specs=[
            pl.BlockSpec(memory_space=pl.ANY),
            pl.BlockSpec(memory_space=pl.ANY),
        ],
        out_specs=pl.BlockSpec(memory_space=pltpu.VMEM),
        scratch_shapes=[
            pltpu.VMEM((m, n), jnp.float32),
            pltpu.VMEM((NZ, 1, m), jnp.int32),
            pltpu.VMEM((NZ, CAP, n), jnp.bfloat16),
            pltpu.VMEM((NZ, CAP, n), jnp.bfloat16),
            pltpu.SemaphoreType.DMA((3,)),
            pltpu.SemaphoreType.DMA((NZ - 1,)),
            pltpu.SemaphoreType.DMA((NZ,)),
            pltpu.SemaphoreType.DMA((NZ - 1,)),
            pltpu.SemaphoreType.DMA((NZ,)),
        ],
        compiler_params=pltpu.CompilerParams(
            collective_id=0 if COMM else None
        ),
    )(x, dest2)
